# Initial kernel scaffold; baseline (speedup 1.0000x reference)
import jax
import jax.numpy as jnp
from jax.experimental import pallas as pl


def kernel(x, edge_index, edge_attr, batch, cond, params):
    raise NotImplementedError("write your pallas kernel here")


# speedup vs baseline: 1.0005x; 1.0005x over previous
import functools

import jax
import jax.numpy as jnp
import numpy as np
from jax.experimental import pallas as pl

N_NODES = 10000
N_EDGES = 320000
N_GRAPHS = 100
NUM_EMB = 64
NUM_HEADS = 2
PRIOR_WEIGHT = 1.0


def _rup(n, m):
    return (n + m - 1) // m * m


def _lin_kernel(x_ref, w_ref, b_ref, o_ref, *, act):
    y = jnp.dot(x_ref[...], w_ref[...], preferred_element_type=jnp.float32)
    y = y + b_ref[...]
    if act:
        y = jnp.where(y > 0, y, 0.01 * y)
    o_ref[...] = y


@functools.partial(jax.jit, static_argnames=("act",))
def _pl_lin(x, W, b, act=False):
    M, K = x.shape
    N = W.shape[1]
    BM = 1024
    Mp, Kp, Np = _rup(M, BM), _rup(K, 128), _rup(N, 128)
    xp = jnp.pad(x, ((0, Mp - M), (0, Kp - K)))
    Wp = jnp.pad(W, ((0, Kp - K), (0, Np - N)))
    bp = jnp.pad(b, (0, Np - N)) if b is not None else jnp.zeros((Np,), jnp.float32)
    out = pl.pallas_call(
        functools.partial(_lin_kernel, act=act),
        grid=(Mp // BM,),
        in_specs=[
            pl.BlockSpec((BM, Kp), lambda i: (i, 0)),
            pl.BlockSpec((Kp, Np), lambda i: (0, 0)),
            pl.BlockSpec((Np,), lambda i: (0,)),
        ],
        out_specs=pl.BlockSpec((BM, Np), lambda i: (i, 0)),
        out_shape=jax.ShapeDtypeStruct((Mp, Np), jnp.float32),
    )(xp, Wp, bp)
    return out[:M, :N]


def _apply_lin(p, x):
    return _pl_lin(x, p["W"], p["b"])


def _apply_mlp(ps, x):
    for i, p in enumerate(ps):
        x = _pl_lin(x, p["W"], p["b"], act=(i < len(ps) - 1))
    return x


def _graph_ln(x, batch, n_graphs):
    F = x.shape[1]
    cnt = jax.ops.segment_sum(jnp.ones((x.shape[0],), jnp.float32), batch, n_graphs)
    mu = jax.ops.segment_sum(x.sum(1), batch, n_graphs) / (cnt * F)
    xc = x - mu[batch][:, None]
    var = jax.ops.segment_sum((xc * xc).sum(1), batch, n_graphs) / (cnt * F)
    return xc * jax.lax.rsqrt(var + 1e-5)[batch][:, None]


def _gt_forward(p, x, edge_index, edge_attr, batch, cond):
    N = x.shape[0]
    G = cond.shape[0]
    o = _apply_mlp(p["x2h"], x)
    e = _apply_mlp(p["e2h"], edge_attr)
    c = _apply_mlp(p["c2h"], cond)
    u = jnp.arange(N, dtype=edge_index.dtype)
    v = batch.astype(edge_index.dtype) + N
    aug_ei = jnp.concatenate([edge_index, jnp.stack([u, v]), jnp.stack([v, u])], 1)
    e_p = jnp.zeros((2 * N, NUM_EMB), jnp.float32).at[:, 0].set(1.0)
    aug_e = jnp.concatenate([e, e_p], 0)
    Nt = N + G
    dst0 = aug_ei[1]
    deg = jnp.clip(jax.ops.segment_sum(jnp.ones((dst0.shape[0],), jnp.float32), dst0, Nt), 1.0, None)
    loop_attr = jax.ops.segment_sum(aug_e, dst0, Nt) / deg[:, None]
    loops = jnp.arange(Nt, dtype=edge_index.dtype)
    aug_ei = jnp.concatenate([aug_ei, jnp.stack([loops, loops])], 1)
    aug_e = jnp.concatenate([aug_e, loop_attr], 0)
    aug_batch = jnp.concatenate([batch, jnp.arange(G, dtype=batch.dtype)], 0)
    o = jnp.concatenate([o, c], 0)
    src, dst = aug_ei[0], aug_ei[1]
    d = NUM_EMB
    H = NUM_HEADS
    for layer in p["layers"]:
        cs = _apply_lin(layer["cscale"], c)[aug_batch]
        o_norm = _graph_ln(o, aug_batch, G)
        m = jax.nn.relu(o_norm[src] + aug_e) + 1e-7
        agg = _apply_lin(layer["gen"], jax.ops.segment_sum(m, dst, Nt) + o_norm)
        h = jnp.concatenate([o_norm, agg], 1)
        q = _apply_lin(layer["q"], h)[dst].reshape(-1, H, d)
        k = _apply_lin(layer["k"], h)[src].reshape(-1, H, d)
        vv = _apply_lin(layer["v"], h)[src].reshape(-1, H, d)
        ee = _pl_lin(aug_e, layer["e"]["W"], None).reshape(-1, H, d)
        alpha = (q * (k + ee)).sum(-1) / np.sqrt(d)
        amax = jax.ops.segment_max(alpha, dst, Nt)
        ae = jnp.exp(alpha - amax[dst])
        denom = jax.ops.segment_sum(ae, dst, Nt)
        attn = ae / jnp.clip(denom[dst], 1e-16, None)
        t_out = jax.ops.segment_sum(attn[..., None] * (vv + ee), dst, Nt).reshape(Nt, H * d)
        t_out = t_out + _apply_lin(layer["skip"], h)
        l_h = _apply_lin(layer["lin"], t_out)
        scale, shift = cs[:, :d], cs[:, d:]
        o = o + l_h * scale + shift
        o = o + _apply_mlp(layer["ff"], _graph_ln(o, aug_batch, G))
    cnt = jnp.clip(jax.ops.segment_sum(jnp.ones((N,), jnp.float32), batch, G), 1.0, None)
    node_mean = jax.ops.segment_sum(o[:N], batch, G) / cnt[:, None]
    glob = jnp.concatenate([node_mean, o[N:]], 1)
    return o[:N], glob


def kernel(x, edge_index, edge_attr, batch, cond, params):
    nm, ng = _gt_forward(params["main"], x, edge_index, edge_attr, batch, cond)
    pn, pg = _gt_forward(params["prior"], x, edge_index, edge_attr, batch, cond)
    ei_nd = edge_index[:, ::2]

    def ef(ne):
        return jnp.concatenate([ne[ei_nd[0]], ne[ei_nd[1]]], 1)

    mh = params["main_heads"]
    ph = params["prior_heads"]
    w = PRIOR_WEIGHT

    def combine(name, m_in, p_in):
        return _apply_mlp(mh[name], m_in) + w * _apply_mlp(ph[name], p_in)

    add_edge = combine("add_edge", ef(nm), ef(pn))
    add_node = combine("add_node", nm, pn)
    set_node_attr = combine("set_node_attr", nm, pn)
    set_edge_attr = combine("set_edge_attr", ef(nm), ef(pn))
    stop = combine("stop", ng, pg)
    reward = _apply_mlp(params["reward"], ng)
    logZ = _apply_mlp(params["logZ"], cond)
    return (add_edge, add_node, set_node_attr, set_edge_attr, stop, reward, logZ)
